# baseline (speedup 1.0000x reference)
import functools

import jax
import jax.numpy as jnp
from jax import lax
from jax.experimental import pallas as pl
from jax.experimental.pallas import tpu as pltpu
from jax.experimental.pallas import tpu_sc as plsc


def _build_kernel(N, D, n_per_w, n_chunks, C, num_cores):
    mesh = plsc.VectorSubcoreMesh(core_axis_name="c", subcore_axis_name="s")

    @functools.partial(
        pl.kernel,
        mesh=mesh,
        out_type=jax.ShapeDtypeStruct((N, D), jnp.float32),
        scratch_types=[
            pltpu.VMEM((n_chunks, C), jnp.int32),
            pltpu.VMEM((2, C, D), jnp.float32),
            pltpu.SemaphoreType.DMA,
            pltpu.SemaphoreType.DMA,
            pltpu.SemaphoreType.DMA,
            pltpu.SemaphoreType.DMA,
        ],
    )
    def emb_kernel(
        ids_hbm, tab_hbm, out_hbm, idx_v, rows_v, gsem0, gsem1, ssem0, ssem1
    ):
        wid = lax.axis_index("s") * num_cores + lax.axis_index("c")
        base = wid * n_per_w

        pltpu.sync_copy(ids_hbm.at[wid], idx_v)

        gsems = (gsem0, gsem1)
        ssems = (ssem0, ssem1)

        def gather(ci, b):
            return pltpu.make_async_copy(
                tab_hbm.at[idx_v.at[ci]], rows_v.at[b], gsems[b]
            )

        def scatter(ci, b):
            return pltpu.make_async_copy(
                rows_v.at[b], out_hbm.at[pl.ds(base + ci * C, C)], ssems[b]
            )

        gather(0, 0).start()

        def body(i):
            for b in range(2):
                ci = i + b
                gather(ci, b).wait()
                scatter(ci, b).start()

                @pl.when(ci >= 1)
                def _():
                    scatter(ci - 1, 1 - b).wait()

                @pl.when(ci + 1 < n_chunks)
                def _():
                    gather(ci + 1, 1 - b).start()

        pl.loop(0, n_chunks, step=2)(body)
        scatter(n_chunks - 1, (n_chunks - 1) % 2).wait()

    return emb_kernel


def kernel(input_ids, embed_table):
    B, S = input_ids.shape
    V, D = embed_table.shape
    N = B * S

    info = plsc.get_sparse_core_info()
    NW = info.num_cores * info.num_subcores
    assert N % NW == 0
    n_per_w = N // NW
    C = 32
    assert n_per_w % C == 0
    n_chunks = n_per_w // C
    assert n_chunks % 2 == 0

    ids = input_ids.reshape(NW, n_chunks, C).astype(jnp.int32)
    emb_kernel = _build_kernel(N, D, n_per_w, n_chunks, C, info.num_cores)
    out = emb_kernel(ids, embed_table)
    return out.reshape(B, S, D)


# speedup vs baseline: 1.0382x; 1.0382x over previous
import functools

import jax
import jax.numpy as jnp
from jax import lax
from jax.experimental import pallas as pl
from jax.experimental.pallas import tpu as pltpu
from jax.experimental.pallas import tpu_sc as plsc


def _build_kernel(N, D, n_per_w, n_chunks, C, num_cores):
    mesh = plsc.VectorSubcoreMesh(core_axis_name="c", subcore_axis_name="s")

    @functools.partial(
        pl.kernel,
        mesh=mesh,
        out_type=jax.ShapeDtypeStruct((N, D), jnp.float32),
        scratch_types=[
            pltpu.VMEM((n_chunks, C), jnp.int32),
            pltpu.VMEM((2, C, D), jnp.float32),
            pltpu.SemaphoreType.DMA,
            pltpu.SemaphoreType.DMA,
        ],
    )
    def emb_kernel(ids_hbm, tab_hbm, out_hbm, idx_v, rows_v, gsem0, gsem1):
        wid = lax.axis_index("s") * num_cores + lax.axis_index("c")
        base = wid * n_per_w

        pltpu.sync_copy(ids_hbm.at[wid], idx_v)

        gsems = (gsem0, gsem1)

        def gather(ci, b):
            return pltpu.make_async_copy(
                tab_hbm.at[idx_v.at[ci]], rows_v.at[b], gsems[b]
            )

        gather(0, 0).start()

        def body(i):
            for b in range(2):
                ci = i + b
                gather(ci + 1, 1 - b).start()
                gather(ci, b).wait()
                pltpu.sync_copy(
                    rows_v.at[b], out_hbm.at[pl.ds(base + ci * C, C)]
                )

        pl.loop(0, n_chunks - 2, step=2)(body)

        for ci in (n_chunks - 2, n_chunks - 1):
            b = ci % 2
            if ci + 1 < n_chunks:
                gather(ci + 1, 1 - b).start()
            gather(ci, b).wait()
            pltpu.sync_copy(rows_v.at[b], out_hbm.at[pl.ds(base + ci * C, C)])

    return emb_kernel


def kernel(input_ids, embed_table):
    B, S = input_ids.shape
    V, D = embed_table.shape
    N = B * S

    info = plsc.get_sparse_core_info()
    NW = info.num_cores * info.num_subcores
    assert N % NW == 0
    n_per_w = N // NW
    C = 32
    assert n_per_w % C == 0
    n_chunks = n_per_w // C
    assert n_chunks % 2 == 0

    ids = input_ids.reshape(NW, n_chunks, C).astype(jnp.int32)
    emb_kernel = _build_kernel(N, D, n_per_w, n_chunks, C, info.num_cores)
    out = emb_kernel(ids, embed_table)
    return out.reshape(B, S, D)


# speedup vs baseline: 1.0442x; 1.0057x over previous
import functools

import jax
import jax.numpy as jnp
from jax import lax
from jax.experimental import pallas as pl
from jax.experimental.pallas import tpu as pltpu
from jax.experimental.pallas import tpu_sc as plsc


def _build_kernel(N, D, n_chunks, C, num_cores, nw):
    mesh = plsc.VectorSubcoreMesh(core_axis_name="c", subcore_axis_name="s")

    @functools.partial(
        pl.kernel,
        mesh=mesh,
        out_type=jax.ShapeDtypeStruct((N, D), jnp.float32),
        scratch_types=[
            pltpu.VMEM((n_chunks, C), jnp.int32),
            pltpu.VMEM((2, C, D), jnp.float32),
            pltpu.SemaphoreType.DMA,
            pltpu.SemaphoreType.DMA,
        ],
    )
    def emb_kernel(ids_hbm, tab_hbm, out_hbm, idx_v, rows_v, gsem0, gsem1):
        wid = lax.axis_index("s") * num_cores + lax.axis_index("c")
        base = wid * C
        stride = nw * C

        pltpu.sync_copy(ids_hbm.at[wid], idx_v)

        gsems = (gsem0, gsem1)

        def gather(ci, b):
            return pltpu.make_async_copy(
                tab_hbm.at[idx_v.at[ci]], rows_v.at[b], gsems[b]
            )

        gather(0, 0).start()

        def body(i):
            for b in range(2):
                ci = i + b
                gather(ci + 1, 1 - b).start()
                gather(ci, b).wait()
                pltpu.sync_copy(
                    rows_v.at[b], out_hbm.at[pl.ds(base + ci * stride, C)]
                )

        pl.loop(0, n_chunks - 2, step=2)(body)

        for ci in (n_chunks - 2, n_chunks - 1):
            b = ci % 2
            if ci + 1 < n_chunks:
                gather(ci + 1, 1 - b).start()
            gather(ci, b).wait()
            pltpu.sync_copy(rows_v.at[b], out_hbm.at[pl.ds(base + ci * stride, C)])

    return emb_kernel


def kernel(input_ids, embed_table):
    B, S = input_ids.shape
    V, D = embed_table.shape
    N = B * S

    info = plsc.get_sparse_core_info()
    NW = info.num_cores * info.num_subcores
    assert N % NW == 0
    n_per_w = N // NW
    C = 32
    assert n_per_w % C == 0
    n_chunks = n_per_w // C
    assert n_chunks % 2 == 0

    ids = (
        input_ids.reshape(n_chunks, NW, C)
        .transpose(1, 0, 2)
        .astype(jnp.int32)
    )
    emb_kernel = _build_kernel(N, D, n_chunks, C, info.num_cores, NW)
    out = emb_kernel(ids, embed_table)
    return out.reshape(B, S, D)
